# baseline (speedup 1.0000x reference)
import functools

import jax
import jax.numpy as jnp
import numpy as np
from jax import lax
from jax.experimental import pallas as pl
from jax.experimental.pallas import tpu as pltpu
from jax.experimental.pallas import tpu_sc as plsc

_MAX_SEQ_LEN = 2048


def _make_gather(V, D, B):
    info = plsc.get_sparse_core_info()
    NC, NS, L = info.num_cores, info.num_subcores, info.num_lanes
    NW = NC * NS
    b_per_w = B // NW
    n_chunks = b_per_w // L
    half = b_per_w // 2
    assert B % (8 * NW) == 0 and b_per_w % (2 * L) == 0
    delta = np.float32(V - 1) / np.float32(B - 1)
    mesh = plsc.VectorSubcoreMesh(core_axis_name="c", subcore_axis_name="s")

    @functools.partial(
        pl.kernel,
        mesh=mesh,
        out_type=jax.ShapeDtypeStruct((B, D), jnp.float32),
        scratch_types=[
            pltpu.VMEM((b_per_w, D), jnp.float32),
            pltpu.SemaphoreType.DMA,
            pltpu.SemaphoreType.DMA,
            pltpu.SemaphoreType.DMA,
        ],
    )
    def gather_kernel(table_hbm, out_hbm, rows_v, sem_a, sem_b, sem_w):
        wid = lax.axis_index("s") * NC + lax.axis_index("c")
        base = wid * b_per_w
        gathers = []
        for j in range(n_chunks):
            i_vec = base + j * L + lax.iota(jnp.int32, L)
            r_vec = (i_vec.astype(jnp.float32) * delta).astype(jnp.int32)
            sem = sem_a if j * L < half else sem_b
            gathers.append(
                pltpu.async_copy(table_hbm.at[r_vec], rows_v.at[pl.ds(j * L, L)], sem)
            )
        for g in gathers[: n_chunks // 2]:
            g.wait()
        w0 = pltpu.async_copy(
            rows_v.at[pl.ds(0, half)], out_hbm.at[pl.ds(base, half)], sem_w
        )
        for g in gathers[n_chunks // 2 :]:
            g.wait()
        w1 = pltpu.async_copy(
            rows_v.at[pl.ds(half, half)], out_hbm.at[pl.ds(base + half, half)], sem_w
        )
        w0.wait()
        w1.wait()

    return gather_kernel


def kernel(feature):
    T, D = feature.shape
    return _make_gather(T, D, _MAX_SEQ_LEN)(feature)


# speedup vs baseline: 1.0074x; 1.0074x over previous
import functools

import jax
import jax.numpy as jnp
import numpy as np
from jax import lax
from jax.experimental import pallas as pl
from jax.experimental.pallas import tpu as pltpu
from jax.experimental.pallas import tpu_sc as plsc

_MAX_SEQ_LEN = 2048


def _make_gather(V, D, B):
    info = plsc.get_sparse_core_info()
    NC, NS, L = info.num_cores, info.num_subcores, info.num_lanes
    NW = NC * NS
    b_per_w = B // NW
    half = b_per_w // 2
    assert B % (8 * NW) == 0 and b_per_w % (2 * L) == 0
    delta = np.float32(V - 1) / np.float32(B - 1)
    mesh = plsc.VectorSubcoreMesh(core_axis_name="c", subcore_axis_name="s")

    @functools.partial(
        pl.kernel,
        mesh=mesh,
        out_type=jax.ShapeDtypeStruct((B, D), jnp.float32),
        scratch_types=[
            pltpu.VMEM((b_per_w,), jnp.int32),
            pltpu.VMEM((b_per_w, D), jnp.float32),
            pltpu.SemaphoreType.DMA,
            pltpu.SemaphoreType.DMA,
            pltpu.SemaphoreType.DMA,
        ],
    )
    def gather_kernel(table_hbm, out_hbm, idx_v, rows_v, sem_a, sem_b, sem_w):
        wid = lax.axis_index("s") * NC + lax.axis_index("c")
        base = wid * b_per_w
        for j in range(b_per_w // L):
            i_vec = base + j * L + lax.iota(jnp.int32, L)
            r_vec = (i_vec.astype(jnp.float32) * delta).astype(jnp.int32)
            idx_v[pl.ds(j * L, L)] = r_vec
        g0 = pltpu.async_copy(
            table_hbm.at[idx_v.at[pl.ds(0, half)]], rows_v.at[pl.ds(0, half)], sem_a
        )
        g1 = pltpu.async_copy(
            table_hbm.at[idx_v.at[pl.ds(half, half)]],
            rows_v.at[pl.ds(half, half)],
            sem_b,
        )
        g0.wait()
        w0 = pltpu.async_copy(
            rows_v.at[pl.ds(0, half)], out_hbm.at[pl.ds(base, half)], sem_w
        )
        g1.wait()
        w1 = pltpu.async_copy(
            rows_v.at[pl.ds(half, half)], out_hbm.at[pl.ds(base + half, half)], sem_w
        )
        w0.wait()
        w1.wait()

    return gather_kernel


def kernel(feature):
    T, D = feature.shape
    return _make_gather(T, D, _MAX_SEQ_LEN)(feature)
